# baseline (speedup 1.0000x reference)
import functools

import jax
import jax.numpy as jnp
from jax import lax
from jax.experimental import pallas as pl
from jax.experimental.pallas import tpu as pltpu
from jax.experimental.pallas import tpu_sc as plsc

TOPK = 5
NROWS = 4096
NCOLS = 100000
LANES = 16
CHUNK = 128
NWORKERS = 32
ROWS_PER_W = NROWS // NWORKERS
BR = 256
BC = 4096


def _sc_gather_body(tbl_hbm, y_hbm, v_hbm, y_v, idx_v, chunks_v, sem):
    wid = lax.axis_index("s") * 2 + lax.axis_index("c")
    base = wid * ROWS_PER_W
    pltpu.sync_copy(y_hbm.at[pl.ds(base, ROWS_PER_W)], y_v)
    iota = lax.iota(jnp.int32, LANES)
    for g in range(ROWS_PER_W // LANES):
        yg = y_v[pl.ds(g * LANES, LANES)]
        rows = base + g * LANES + iota
        idx_v[pl.ds(g * LANES, LANES)] = (rows * NCOLS + yg) >> 7
    pltpu.async_copy(tbl_hbm.at[idx_v], chunks_v, sem).wait()
    pltpu.sync_copy(chunks_v, v_hbm.at[pl.ds(base, ROWS_PER_W)])


@functools.cache
def _sc_gather_kernel():
    return pl.kernel(
        _sc_gather_body,
        mesh=plsc.VectorSubcoreMesh(core_axis_name="c", subcore_axis_name="s"),
        out_type=jax.ShapeDtypeStruct((NROWS, CHUNK), jnp.float32),
        scratch_types=[
            pltpu.VMEM((ROWS_PER_W,), jnp.int32),
            pltpu.VMEM((ROWS_PER_W,), jnp.int32),
            pltpu.VMEM((ROWS_PER_W, CHUNK), jnp.float32),
            pltpu.SemaphoreType.DMA,
        ],
    )


def _tc_body(x_ref, c_ref, y_ref, out_ref, acc_ref):
    j = pl.program_id(1)

    @pl.when(j == 0)
    def _init():
        acc_ref[...] = jnp.zeros_like(acc_ref)

    x = x_ref[...]
    yv = y_ref[...]
    rows = pl.program_id(0) * BR + lax.broadcasted_iota(jnp.int32, (BR, 1), 0)
    off = (rows * (NCOLS % 128) + yv) & 127
    lane = lax.broadcasted_iota(jnp.int32, (BR, CHUNK), 1)
    picked = jnp.where(lane == off, c_ref[...], 0.0)
    v = picked.sum(axis=1, keepdims=True)
    cols = j * BC + lax.broadcasted_iota(jnp.int32, (BR, BC), 1)
    m = (x > v) | ((x == v) & (cols < yv))
    m = m & (cols < NCOLS)
    ones = jnp.where(m, 1.0, 0.0)
    acc_ref[...] += ones.reshape(BR, BC // 128, 128).sum(axis=1)

    @pl.when(j == pl.num_programs(1) - 1)
    def _fin():
        rank = acc_ref[...].sum(axis=1, keepdims=True)
        match = jnp.where(rank < (TOPK - 0.5), 1.0, 0.0)
        part = jnp.sum(match, keepdims=True).reshape(1, 1)
        i = pl.program_id(0)
        prev = jnp.where(i == 0, jnp.zeros_like(part), out_ref[...])
        scale = jnp.where(i == pl.num_programs(0) - 1, 1.0 / NROWS, 1.0)
        out_ref[...] = (prev + part) * scale


def kernel(y_hat_logits, y):
    y32 = y.astype(jnp.int32)
    tbl = y_hat_logits.reshape(NROWS * NCOLS // CHUNK, CHUNK)
    chunks = _sc_gather_kernel()(tbl, y32)
    out = pl.pallas_call(
        _tc_body,
        grid=(NROWS // BR, pl.cdiv(NCOLS, BC)),
        in_specs=[
            pl.BlockSpec((BR, BC), lambda i, j: (i, j)),
            pl.BlockSpec((BR, CHUNK), lambda i, j: (i, 0)),
            pl.BlockSpec((BR, 1), lambda i, j: (i, 0)),
        ],
        out_specs=pl.BlockSpec((1, 1), lambda i, j: (0, 0)),
        out_shape=jax.ShapeDtypeStruct((1, 1), jnp.float32),
        scratch_shapes=[pltpu.VMEM((BR, 128), jnp.float32)],
    )(y_hat_logits, chunks, y32.reshape(NROWS, 1))
    return out[0, 0]
